# Initial kernel scaffold; baseline (speedup 1.0000x reference)
import jax
import jax.numpy as jnp
from jax.experimental import pallas as pl


def kernel(idx, embedding_table):
    raise NotImplementedError("write your pallas kernel here")


# speedup vs baseline: 2.9684x; 2.9684x over previous
import functools

import jax
import jax.numpy as jnp
from jax import lax
from jax.experimental import pallas as pl
from jax.experimental.pallas import tpu as pltpu
from jax.experimental.pallas import tpu_sc as plsc

VOCAB = 100000
DIM = 128
BATCH = 4096
SEQ = 50

NC = 2
NS = 16
NW = NC * NS
B_TOTAL = BATCH * SEQ
BPW = B_TOTAL // NW
CHUNK = 128
NCHUNK = BPW // CHUNK

_mesh = plsc.VectorSubcoreMesh(core_axis_name="c", subcore_axis_name="s")


@functools.partial(
    pl.kernel,
    out_type=jax.ShapeDtypeStruct((B_TOTAL, DIM), jnp.float32),
    mesh=_mesh,
    scratch_types=[
        pltpu.VMEM((NCHUNK, CHUNK), jnp.int32),
        pltpu.VMEM((CHUNK, DIM), jnp.float32),
        pltpu.SemaphoreType.DMA,
    ],
)
def _gather_kernel(idx_hbm, table_hbm, out_hbm, idx_v, rows_v, gsem):
    wid = lax.axis_index("s") * NC + lax.axis_index("c")
    base = wid * BPW
    pltpu.sync_copy(idx_hbm.at[wid], idx_v)

    def body(j, carry):
        pltpu.async_copy(table_hbm.at[idx_v.at[j]], rows_v, gsem).wait()
        pltpu.sync_copy(rows_v, out_hbm.at[pl.ds(base + j * CHUNK, CHUNK)])
        return carry

    lax.fori_loop(0, NCHUNK, body, 0)


def kernel(idx, embedding_table):
    idx_grouped = idx.reshape(NW, NCHUNK, CHUNK)
    out = _gather_kernel(idx_grouped, embedding_table)
    return out.reshape(BATCH, SEQ, DIM)


# speedup vs baseline: 3.3005x; 1.1119x over previous
import functools

import jax
import jax.numpy as jnp
from jax import lax
from jax.experimental import pallas as pl
from jax.experimental.pallas import tpu as pltpu
from jax.experimental.pallas import tpu_sc as plsc

VOCAB = 100000
DIM = 128
BATCH = 4096
SEQ = 50

NC = 2
NS = 16
NW = NC * NS
B_TOTAL = BATCH * SEQ
BPW = B_TOTAL // NW
CHUNK = 128
NCHUNK = BPW // CHUNK
NBUF = 5
NGROUP = NCHUNK // NBUF

_mesh = plsc.VectorSubcoreMesh(core_axis_name="c", subcore_axis_name="s")


@functools.partial(
    pl.kernel,
    out_type=jax.ShapeDtypeStruct((B_TOTAL, DIM), jnp.float32),
    mesh=_mesh,
    scratch_types=[
        pltpu.VMEM((NCHUNK, CHUNK), jnp.int32),
        [pltpu.VMEM((CHUNK, DIM), jnp.float32) for _ in range(NBUF)],
        [pltpu.SemaphoreType.DMA for _ in range(NBUF)],
        [pltpu.SemaphoreType.DMA for _ in range(NBUF)],
    ],
)
def _gather_kernel(idx_hbm, table_hbm, out_hbm, idx_v, bufs, gsems, ssems):
    wid = lax.axis_index("s") * NC + lax.axis_index("c")
    base = wid * BPW
    pltpu.sync_copy(idx_hbm.at[wid], idx_v)

    def fire_gather(c, b):
        return pltpu.async_copy(table_hbm.at[idx_v.at[c]], bufs[b], gsems[b])

    def fire_scatter(c, b):
        return pltpu.async_copy(bufs[b], out_hbm.at[pl.ds(base + c * CHUNK, CHUNK)], ssems[b])

    def wait_scatter(b):
        pltpu.make_async_copy(bufs[b], out_hbm.at[pl.ds(base, CHUNK)], ssems[b]).wait()

    gds = [fire_gather(b, b) for b in range(NBUF)]
    for b in range(NBUF):
        gds[b].wait()
        fire_scatter(b, b)

    def body(g, carry):
        gds = []
        for b in range(NBUF):
            wait_scatter(b)
            gds.append(fire_gather(g * NBUF + b, b))
        for b in range(NBUF):
            gds[b].wait()
            fire_scatter(g * NBUF + b, b)
        return carry

    lax.fori_loop(1, NGROUP, body, 0)

    for b in range(NBUF):
        wait_scatter(b)


def kernel(idx, embedding_table):
    idx_grouped = idx.reshape(NW, NCHUNK, CHUNK)
    out = _gather_kernel(idx_grouped, embedding_table)
    return out.reshape(BATCH, SEQ, DIM)


# speedup vs baseline: 5.7753x; 1.7498x over previous
import functools

import jax
import jax.numpy as jnp
from jax import lax
from jax.experimental import pallas as pl
from jax.experimental.pallas import tpu as pltpu
from jax.experimental.pallas import tpu_sc as plsc

VOCAB = 100000
DIM = 128
BATCH = 4096
SEQ = 50

NC = 2
NS = 16
NW = NC * NS
RPW = BATCH // NW
NBUF = 4
NGROUP = RPW // NBUF

_mesh = plsc.VectorSubcoreMesh(core_axis_name="c", subcore_axis_name="s")


@functools.partial(
    pl.kernel,
    out_type=jax.ShapeDtypeStruct((BATCH, SEQ, DIM), jnp.float32),
    mesh=_mesh,
    scratch_types=[
        pltpu.VMEM((RPW, SEQ), jnp.int32),
        [pltpu.VMEM((SEQ, DIM), jnp.float32) for _ in range(NBUF)],
        [pltpu.SemaphoreType.DMA for _ in range(NBUF)],
        [pltpu.SemaphoreType.DMA for _ in range(NBUF)],
    ],
)
def _gather_kernel(idx_hbm, table_hbm, out_hbm, idx_v, bufs, gsems, ssems):
    wid = lax.axis_index("s") * NC + lax.axis_index("c")
    base = wid * RPW
    pltpu.sync_copy(idx_hbm.at[wid], idx_v)

    def fire_gather(r, b):
        return pltpu.async_copy(table_hbm.at[idx_v.at[r]], bufs[b], gsems[b])

    def fire_scatter(r, b):
        return pltpu.async_copy(bufs[b], out_hbm.at[base + r], ssems[b])

    def wait_scatter(b):
        pltpu.make_async_copy(bufs[b], out_hbm.at[base], ssems[b]).wait()

    gds = [fire_gather(b, b) for b in range(NBUF)]
    for b in range(NBUF):
        gds[b].wait()
        fire_scatter(b, b)

    def body(g, carry):
        gds = []
        for b in range(NBUF):
            wait_scatter(b)
            gds.append(fire_gather(g * NBUF + b, b))
        for b in range(NBUF):
            gds[b].wait()
            fire_scatter(g * NBUF + b, b)
        return carry

    lax.fori_loop(1, NGROUP, body, 0)

    for b in range(NBUF):
        wait_scatter(b)


def kernel(idx, embedding_table):
    idx_grouped = idx.reshape(NW, RPW, SEQ)
    return _gather_kernel(idx_grouped, embedding_table)


# speedup vs baseline: 5.8016x; 1.0045x over previous
import functools

import jax
import jax.numpy as jnp
from jax import lax
from jax.experimental import pallas as pl
from jax.experimental.pallas import tpu as pltpu
from jax.experimental.pallas import tpu_sc as plsc

VOCAB = 100000
DIM = 128
BATCH = 4096
SEQ = 50

NC = 2
NS = 16
NW = NC * NS
RPW = BATCH // NW
NBUF = 4
NGROUP = RPW // NBUF

_mesh = plsc.VectorSubcoreMesh(core_axis_name="c", subcore_axis_name="s")


@functools.partial(
    pl.kernel,
    out_type=jax.ShapeDtypeStruct((BATCH, SEQ, DIM), jnp.float32),
    mesh=_mesh,
    scratch_types=[
        pltpu.VMEM((RPW, SEQ), jnp.int32),
        [pltpu.VMEM((SEQ, DIM), jnp.float32) for _ in range(NBUF)],
        [pltpu.SemaphoreType.DMA for _ in range(NBUF)],
        [pltpu.SemaphoreType.DMA for _ in range(NBUF)],
    ],
)
def _gather_kernel(idx_hbm, table_hbm, out_hbm, idx_v, bufs, gsems, ssems):
    wid = lax.axis_index("s") * NC + lax.axis_index("c")
    base = wid * RPW
    pltpu.sync_copy(idx_hbm.at[pl.ds(base, RPW)], idx_v)

    def fire_gather(r, b):
        return pltpu.async_copy(table_hbm.at[idx_v.at[r]], bufs[b], gsems[b])

    def fire_scatter(r, b):
        return pltpu.async_copy(bufs[b], out_hbm.at[base + r], ssems[b])

    def wait_scatter(b):
        pltpu.make_async_copy(bufs[b], out_hbm.at[base], ssems[b]).wait()

    gds = [fire_gather(b, b) for b in range(NBUF)]
    for b in range(NBUF):
        gds[b].wait()
        fire_scatter(b, b)

    def body(g, carry):
        gds = []
        for b in range(NBUF):
            wait_scatter(b)
            gds.append(fire_gather(g * NBUF + b, b))
        for b in range(NBUF):
            gds[b].wait()
            fire_scatter(g * NBUF + b, b)
        return carry

    lax.fori_loop(1, NGROUP, body, 0)

    for b in range(NBUF):
        wait_scatter(b)


def kernel(idx, embedding_table):
    return _gather_kernel(idx, embedding_table)
